# baseline (speedup 1.0000x reference)
import functools

import jax
import jax.numpy as jnp
from jax import lax
from jax.experimental import pallas as pl
from jax.experimental.pallas import tpu as pltpu
from jax.experimental.pallas import tpu_sc as plsc
from jax.experimental import layout as jex_layout

B = 16384
D = 64
NC = 2
NS = 16
NW = NC * NS
BPW = B // NW
H = 256
L = 16


def _row_gather_kernel(idx, table):
    mesh = plsc.VectorSubcoreMesh(
        core_axis_name="c", subcore_axis_name="s", num_cores=NC, num_subcores=NS
    )

    @functools.partial(
        pl.kernel,
        out_type=jax.ShapeDtypeStruct((B, D), jnp.float32),
        mesh=mesh,
        scratch_types=[
            pltpu.VMEM((BPW + L,), jnp.int32),
            pltpu.VMEM((H, D), jnp.float32),
            pltpu.SemaphoreType.DMA,
        ],
    )
    def k(idx_h, tab_h, out_o, idx_v, ob_v, sem):
        wid = lax.axis_index("s") * NC + lax.axis_index("c")
        base = wid * BPW
        pltpu.sync_copy(idx_h.at[pl.ds(base, BPW)], idx_v.at[pl.ds(0, BPW)])

        for h in range(BPW // H):
            def body(g, _):
                v = idx_v[pl.ds(h * H + g * L, L)]
                for j in range(L):
                    pltpu.async_copy(
                        tab_h.at[pl.ds(v[j], 1)],
                        ob_v.at[pl.ds(g * L + j, 1)], sem)
                return _

            lax.fori_loop(0, H // L, body, None)
            pltpu.make_async_copy(tab_h.at[pl.ds(0, H)], ob_v, sem).wait()
            pltpu.sync_copy(ob_v, out_o.at[pl.ds(base + h * H, H)])

    return k(idx, table)


def _sc_gather_bias(item, Bi):
    mesh = plsc.VectorSubcoreMesh(
        core_axis_name="c", subcore_axis_name="s", num_cores=NC, num_subcores=NS
    )

    @functools.partial(
        pl.kernel,
        out_type=jax.ShapeDtypeStruct((B,), jnp.float32),
        mesh=mesh,
        scratch_types=[
            pltpu.VMEM((BPW,), jnp.int32),
            pltpu.VMEM((BPW,), jnp.float32),
            pltpu.SemaphoreType.DMA,
        ],
        compiler_params=pltpu.CompilerParams(use_tc_tiling_on_sc=False),
    )
    def k(item_h, bi_h, beta_o, iidx_v, beta_v, sem):
        wid = lax.axis_index("s") * NC + lax.axis_index("c")
        base = wid * BPW
        pltpu.sync_copy(item_h.at[pl.ds(base, BPW)], iidx_v)
        pltpu.async_copy(bi_h.at[iidx_v], beta_v, sem).wait()
        pltpu.sync_copy(beta_v, beta_o.at[pl.ds(base, BPW)])

    return k(item, Bi)


def _dot_body(beta_ref, gu_ref, gi_ref, out_ref):
    out_ref[...] = beta_ref[...] + jnp.sum(gu_ref[...] * gi_ref[...], axis=1)


def _tc_dot(beta, gu, gi):
    return pl.pallas_call(
        _dot_body,
        out_shape=jax.ShapeDtypeStruct((B,), jnp.float32),
    )(beta, gu, gi)


def kernel(user, item, Bi, Gu, Gi):
    fmt = jex_layout.Layout(major_to_minor=(0, 1))
    Gi_rm = jex_layout.with_layout_constraint(Gi, fmt)
    Gi_rm = lax.optimization_barrier(Gi_rm)
    Gu_rm = Gu
    gamma_u = _row_gather_kernel(user, Gu_rm)
    gamma_i = _row_gather_kernel(item, Gi_rm)
    beta_i = _sc_gather_bias(item, Bi)
    xui = _tc_dot(beta_i, gamma_u, gamma_i)
    return (xui, beta_i, gamma_u, gamma_i)


# speedup vs baseline: 1.0798x; 1.0798x over previous
import functools

import jax
import jax.numpy as jnp
from jax import lax
from jax.experimental import pallas as pl
from jax.experimental.pallas import tpu as pltpu
from jax.experimental.pallas import tpu_sc as plsc
from jax.experimental import layout as jex_layout

B = 16384
D = 64
NC = 2
NS = 16
NW = NC * NS
BPW = B // NW
H = 256
L = 16


def _row_gather_kernel(user, item, Gu_rm, Gi_rm):
    mesh = plsc.VectorSubcoreMesh(
        core_axis_name="c", subcore_axis_name="s", num_cores=NC, num_subcores=NS
    )

    @functools.partial(
        pl.kernel,
        out_type=[
            jax.ShapeDtypeStruct((B, D), jnp.float32),
            jax.ShapeDtypeStruct((B, D), jnp.float32),
        ],
        mesh=mesh,
        scratch_types=[
            pltpu.VMEM((BPW + L,), jnp.int32),
            pltpu.VMEM((BPW + L,), jnp.int32),
            pltpu.VMEM((H, D), jnp.float32),
            pltpu.VMEM((H, D), jnp.float32),
            pltpu.SemaphoreType.DMA,
            pltpu.SemaphoreType.DMA,
        ],
    )
    def k(user_h, item_h, gu_h, gi_h, gu_o, gi_o,
          uidx_v, iidx_v, obu_v, obi_v, sem_u, sem_i):
        wid = lax.axis_index("s") * NC + lax.axis_index("c")
        base = wid * BPW
        pltpu.sync_copy(user_h.at[pl.ds(base, BPW)], uidx_v.at[pl.ds(0, BPW)])
        pltpu.sync_copy(item_h.at[pl.ds(base, BPW)], iidx_v.at[pl.ds(0, BPW)])

        for h in range(BPW // H):
            def body(g, _):
                vu = uidx_v[pl.ds(h * H + g * L, L)]
                vi = iidx_v[pl.ds(h * H + g * L, L)]
                for j in range(L):
                    pltpu.async_copy(
                        gu_h.at[pl.ds(vu[j], 1)],
                        obu_v.at[pl.ds(g * L + j, 1)], sem_u)
                    pltpu.async_copy(
                        gi_h.at[pl.ds(vi[j], 1)],
                        obi_v.at[pl.ds(g * L + j, 1)], sem_i)
                return _

            lax.fori_loop(0, H // L, body, None)
            pltpu.make_async_copy(gu_h.at[pl.ds(0, H)], obu_v, sem_u).wait()
            pltpu.make_async_copy(gi_h.at[pl.ds(0, H)], obi_v, sem_i).wait()
            pltpu.sync_copy(obu_v, gu_o.at[pl.ds(base + h * H, H)])
            pltpu.sync_copy(obi_v, gi_o.at[pl.ds(base + h * H, H)])

    return k(user, item, Gu_rm, Gi_rm)


def _sc_gather_bias(item, Bi):
    mesh = plsc.VectorSubcoreMesh(
        core_axis_name="c", subcore_axis_name="s", num_cores=NC, num_subcores=NS
    )

    @functools.partial(
        pl.kernel,
        out_type=jax.ShapeDtypeStruct((B,), jnp.float32),
        mesh=mesh,
        scratch_types=[
            pltpu.VMEM((BPW,), jnp.int32),
            pltpu.VMEM((BPW,), jnp.float32),
            pltpu.SemaphoreType.DMA,
        ],
        compiler_params=pltpu.CompilerParams(use_tc_tiling_on_sc=False),
    )
    def k(item_h, bi_h, beta_o, iidx_v, beta_v, sem):
        wid = lax.axis_index("s") * NC + lax.axis_index("c")
        base = wid * BPW
        pltpu.sync_copy(item_h.at[pl.ds(base, BPW)], iidx_v)
        pltpu.async_copy(bi_h.at[iidx_v], beta_v, sem).wait()
        pltpu.sync_copy(beta_v, beta_o.at[pl.ds(base, BPW)])

    return k(item, Bi)


def _dot_body(beta_ref, gu_ref, gi_ref, out_ref):
    out_ref[...] = beta_ref[...] + jnp.sum(gu_ref[...] * gi_ref[...], axis=1)


def _tc_dot(beta, gu, gi):
    return pl.pallas_call(
        _dot_body,
        out_shape=jax.ShapeDtypeStruct((B,), jnp.float32),
    )(beta, gu, gi)


def kernel(user, item, Bi, Gu, Gi):
    fmt = jex_layout.Layout(major_to_minor=(0, 1))
    Gu_rm = jex_layout.with_layout_constraint(Gu, fmt)
    Gi_rm = jex_layout.with_layout_constraint(Gi, fmt)
    Gu_rm, Gi_rm = lax.optimization_barrier((Gu_rm, Gi_rm))
    gamma_u, gamma_i = _row_gather_kernel(user, item, Gu_rm, Gi_rm)
    beta_i = _sc_gather_bias(item, Bi)
    xui = _tc_dot(beta_i, gamma_u, gamma_i)
    return (xui, beta_i, gamma_u, gamma_i)
